# Initial kernel scaffold; baseline (speedup 1.0000x reference)
import jax
import jax.numpy as jnp
from jax.experimental import pallas as pl


def kernel(phoneme_indices, phoneme_table, sutra_table, position_table, sutra_lookup, position_lookup, proj_w, proj_b):
    raise NotImplementedError("write your pallas kernel here")


# speedup vs baseline: 32.5196x; 32.5196x over previous
import functools

import jax
import jax.numpy as jnp
from jax import lax
from jax.experimental import pallas as pl
from jax.experimental.pallas import tpu as pltpu
from jax.experimental.pallas import tpu_sc as plsc

VOCAB = 100000
D = 64
ROWS_PER_BLOCK = 800

NC = 2
NS = 16
NW = NC * NS
CHUNK = 512


def _fused_table_body(ph_ref, su_idx_ref, po_idx_ref, su_tab_ref, po_tab_ref,
                      w_ref, b_ref, out_ref):
    wph = w_ref[0:64, :]
    wsu = w_ref[64:128, :]
    wpo = w_ref[128:192, :]
    sp = jnp.dot(su_tab_ref[...], wsu, preferred_element_type=jnp.float32)
    pp = jnp.dot(po_tab_ref[...], wpo, preferred_element_type=jnp.float32)
    su_idx = su_idx_ref[0, 0, :]
    po_idx = po_idx_ref[0, 0, :]
    lanes = lax.broadcasted_iota(jnp.int32, (ROWS_PER_BLOCK, 16), 1)
    su_oh = (su_idx[:, None] == lanes).astype(jnp.float32)
    po_oh = (po_idx[:, None] == lanes).astype(jnp.float32)
    acc = jnp.dot(ph_ref[...], wph, preferred_element_type=jnp.float32)
    acc += jnp.dot(su_oh, sp, preferred_element_type=jnp.float32)
    acc += jnp.dot(po_oh, pp, preferred_element_type=jnp.float32)
    out_ref[...] = acc + b_ref[...]


def _build_fused_table(phoneme_table, sutra_table, position_table,
                       sutra_lookup, position_lookup, proj_w, proj_b):
    nblk = VOCAB // ROWS_PER_BLOCK
    su_idx = sutra_lookup.astype(jnp.int32).reshape(nblk, 1, ROWS_PER_BLOCK)
    po_idx = position_lookup.astype(jnp.int32).reshape(nblk, 1, ROWS_PER_BLOCK)
    su_tab = jnp.zeros((16, D), jnp.float32).at[:15].set(sutra_table)
    po_tab = jnp.zeros((16, D), jnp.float32).at[:11].set(position_table)
    return pl.pallas_call(
        _fused_table_body,
        grid=(nblk,),
        in_specs=[
            pl.BlockSpec((ROWS_PER_BLOCK, D), lambda i: (i, 0)),
            pl.BlockSpec((1, 1, ROWS_PER_BLOCK), lambda i: (i, 0, 0)),
            pl.BlockSpec((1, 1, ROWS_PER_BLOCK), lambda i: (i, 0, 0)),
            pl.BlockSpec((16, D), lambda i: (0, 0)),
            pl.BlockSpec((16, D), lambda i: (0, 0)),
            pl.BlockSpec((192, D), lambda i: (0, 0)),
            pl.BlockSpec((1, D), lambda i: (0, 0)),
        ],
        out_specs=pl.BlockSpec((ROWS_PER_BLOCK, D), lambda i: (i, 0)),
        out_shape=jax.ShapeDtypeStruct((VOCAB, D), jnp.float32),
    )(phoneme_table, su_idx, po_idx, su_tab, po_tab, proj_w,
      proj_b.reshape(1, D))


def _sc_gather(table, idx, batch):
    b_per_w = batch // NW
    nchunk = b_per_w // CHUNK
    mesh = plsc.VectorSubcoreMesh(core_axis_name="c", subcore_axis_name="s",
                                  num_cores=NC, num_subcores=NS)

    @functools.partial(
        pl.kernel,
        mesh=mesh,
        compiler_params=pltpu.CompilerParams(use_tc_tiling_on_sc=False),
        out_type=jax.ShapeDtypeStruct((batch, D), jnp.float32),
        scratch_types=[
            pltpu.VMEM((CHUNK,), jnp.int32),
            pltpu.VMEM((CHUNK,), jnp.int32),
            pltpu.VMEM((CHUNK, D), jnp.float32),
            pltpu.VMEM((CHUNK, D), jnp.float32),
            pltpu.SemaphoreType.DMA,
            pltpu.SemaphoreType.DMA,
        ],
    )
    def gather_kernel(table_hbm, idx_hbm, out_hbm,
                      idx0, idx1, rows0, rows1, sem0, sem1):
        wid = lax.axis_index("s") * NC + lax.axis_index("c")
        base = wid * b_per_w

        pltpu.sync_copy(idx_hbm.at[pl.ds(base, CHUNK)], idx0)
        pltpu.async_copy(table_hbm.at[idx0], rows0, sem0)
        pltpu.sync_copy(idx_hbm.at[pl.ds(base + CHUNK, CHUNK)], idx1)
        pltpu.async_copy(table_hbm.at[idx1], rows1, sem1)

        def body(p, carry):
            g = 2 * p
            pltpu.make_async_copy(table_hbm.at[idx0], rows0, sem0).wait()
            pltpu.sync_copy(rows0, out_hbm.at[pl.ds(base + g * CHUNK, CHUNK)])
            pltpu.sync_copy(idx_hbm.at[pl.ds(base + (g + 2) * CHUNK, CHUNK)],
                            idx0)
            pltpu.async_copy(table_hbm.at[idx0], rows0, sem0)
            pltpu.make_async_copy(table_hbm.at[idx1], rows1, sem1).wait()
            pltpu.sync_copy(rows1,
                            out_hbm.at[pl.ds(base + (g + 1) * CHUNK, CHUNK)])
            pltpu.sync_copy(idx_hbm.at[pl.ds(base + (g + 3) * CHUNK, CHUNK)],
                            idx1)
            pltpu.async_copy(table_hbm.at[idx1], rows1, sem1)
            return carry

        lax.fori_loop(0, nchunk // 2 - 2, body, 0)

        g = nchunk - 4
        pltpu.make_async_copy(table_hbm.at[idx0], rows0, sem0).wait()
        pltpu.sync_copy(rows0, out_hbm.at[pl.ds(base + g * CHUNK, CHUNK)])
        pltpu.sync_copy(idx_hbm.at[pl.ds(base + (g + 2) * CHUNK, CHUNK)], idx0)
        pltpu.async_copy(table_hbm.at[idx0], rows0, sem0)
        pltpu.make_async_copy(table_hbm.at[idx1], rows1, sem1).wait()
        pltpu.sync_copy(rows1, out_hbm.at[pl.ds(base + (g + 1) * CHUNK, CHUNK)])
        pltpu.sync_copy(idx_hbm.at[pl.ds(base + (g + 3) * CHUNK, CHUNK)], idx1)
        pltpu.async_copy(table_hbm.at[idx1], rows1, sem1)

        pltpu.make_async_copy(table_hbm.at[idx0], rows0, sem0).wait()
        pltpu.sync_copy(rows0,
                        out_hbm.at[pl.ds(base + (g + 2) * CHUNK, CHUNK)])
        pltpu.make_async_copy(table_hbm.at[idx1], rows1, sem1).wait()
        pltpu.sync_copy(rows1,
                        out_hbm.at[pl.ds(base + (g + 3) * CHUNK, CHUNK)])

    return gather_kernel(table, idx)


def kernel(phoneme_indices, phoneme_table, sutra_table, position_table,
           sutra_lookup, position_lookup, proj_w, proj_b):
    batch, seq = phoneme_indices.shape
    fused = _build_fused_table(phoneme_table, sutra_table, position_table,
                               sutra_lookup, position_lookup, proj_w, proj_b)
    idx = phoneme_indices.reshape(-1).astype(jnp.int32)
    out = _sc_gather(fused, idx, batch * seq)
    return out.reshape(batch, seq, D)


# speedup vs baseline: 43.2739x; 1.3307x over previous
import functools

import jax
import jax.numpy as jnp
from jax import lax
from jax.experimental import pallas as pl
from jax.experimental.pallas import tpu as pltpu
from jax.experimental.pallas import tpu_sc as plsc

VOCAB = 100000
D = 64
ROWS_PER_BLOCK = 800

NC = 2
NS = 16
NW = NC * NS
CHUNK = 400


def _fused_table_body(ph_ref, su_idx_ref, po_idx_ref, su_tab_ref, po_tab_ref,
                      w_ref, b_ref, out_ref):
    wph = w_ref[0:64, :]
    wsu = w_ref[64:128, :]
    wpo = w_ref[128:192, :]
    sp = jnp.dot(su_tab_ref[...], wsu, preferred_element_type=jnp.float32)
    pp = jnp.dot(po_tab_ref[...], wpo, preferred_element_type=jnp.float32)
    su_idx = su_idx_ref[0, 0, :]
    po_idx = po_idx_ref[0, 0, :]
    lanes = lax.broadcasted_iota(jnp.int32, (ROWS_PER_BLOCK, 16), 1)
    su_oh = (su_idx[:, None] == lanes).astype(jnp.float32)
    po_oh = (po_idx[:, None] == lanes).astype(jnp.float32)
    acc = jnp.dot(ph_ref[...], wph, preferred_element_type=jnp.float32)
    acc += jnp.dot(su_oh, sp, preferred_element_type=jnp.float32)
    acc += jnp.dot(po_oh, pp, preferred_element_type=jnp.float32)
    acc += b_ref[...]
    out_ref[...] = jnp.concatenate(
        [acc, jnp.zeros((ROWS_PER_BLOCK, D), jnp.float32)], axis=1)


def _build_fused_table(phoneme_table, sutra_table, position_table,
                       sutra_lookup, position_lookup, proj_w, proj_b):
    nblk = VOCAB // ROWS_PER_BLOCK
    su_idx = sutra_lookup.astype(jnp.int32).reshape(nblk, 1, ROWS_PER_BLOCK)
    po_idx = position_lookup.astype(jnp.int32).reshape(nblk, 1, ROWS_PER_BLOCK)
    su_tab = jnp.zeros((16, D), jnp.float32).at[:15].set(sutra_table)
    po_tab = jnp.zeros((16, D), jnp.float32).at[:11].set(position_table)
    return pl.pallas_call(
        _fused_table_body,
        grid=(nblk,),
        in_specs=[
            pl.BlockSpec((ROWS_PER_BLOCK, D), lambda i: (i, 0)),
            pl.BlockSpec((1, 1, ROWS_PER_BLOCK), lambda i: (i, 0, 0)),
            pl.BlockSpec((1, 1, ROWS_PER_BLOCK), lambda i: (i, 0, 0)),
            pl.BlockSpec((16, D), lambda i: (0, 0)),
            pl.BlockSpec((16, D), lambda i: (0, 0)),
            pl.BlockSpec((192, D), lambda i: (0, 0)),
            pl.BlockSpec((1, D), lambda i: (0, 0)),
        ],
        out_specs=pl.BlockSpec((ROWS_PER_BLOCK, 2 * D), lambda i: (i, 0)),
        out_shape=jax.ShapeDtypeStruct((VOCAB, 2 * D), jnp.float32),
    )(phoneme_table, su_idx, po_idx, su_tab, po_tab, proj_w,
      proj_b.reshape(1, D))


def _sc_gather(table, idx, batch):
    b_per_w = batch // NW
    nchunk = b_per_w // CHUNK
    mesh = plsc.VectorSubcoreMesh(core_axis_name="c", subcore_axis_name="s",
                                  num_cores=NC, num_subcores=NS)

    @functools.partial(
        pl.kernel,
        mesh=mesh,
        compiler_params=pltpu.CompilerParams(use_tc_tiling_on_sc=False),
        out_type=jax.ShapeDtypeStruct((batch, 2 * D), jnp.float32),
        scratch_types=[
            pltpu.VMEM((CHUNK,), jnp.int32),
            pltpu.VMEM((CHUNK,), jnp.int32),
            pltpu.VMEM((CHUNK, 2 * D), jnp.float32),
            pltpu.VMEM((CHUNK, 2 * D), jnp.float32),
            pltpu.SemaphoreType.DMA,
            pltpu.SemaphoreType.DMA,
        ],
    )
    def gather_kernel(table_hbm, idx_hbm, out_hbm,
                      idx0, idx1, rows0, rows1, sem0, sem1):
        wid = lax.axis_index("s") * NC + lax.axis_index("c")
        base = wid * b_per_w

        pltpu.sync_copy(idx_hbm.at[pl.ds(base, CHUNK)], idx0)
        pltpu.async_copy(table_hbm.at[idx0], rows0, sem0)
        pltpu.sync_copy(idx_hbm.at[pl.ds(base + CHUNK, CHUNK)], idx1)
        pltpu.async_copy(table_hbm.at[idx1], rows1, sem1)

        def body(p, carry):
            g = 2 * p
            pltpu.make_async_copy(table_hbm.at[idx0], rows0, sem0).wait()
            pltpu.sync_copy(rows0, out_hbm.at[pl.ds(base + g * CHUNK, CHUNK)])
            pltpu.sync_copy(idx_hbm.at[pl.ds(base + (g + 2) * CHUNK, CHUNK)],
                            idx0)
            pltpu.async_copy(table_hbm.at[idx0], rows0, sem0)
            pltpu.make_async_copy(table_hbm.at[idx1], rows1, sem1).wait()
            pltpu.sync_copy(rows1,
                            out_hbm.at[pl.ds(base + (g + 1) * CHUNK, CHUNK)])
            pltpu.sync_copy(idx_hbm.at[pl.ds(base + (g + 3) * CHUNK, CHUNK)],
                            idx1)
            pltpu.async_copy(table_hbm.at[idx1], rows1, sem1)
            return carry

        lax.fori_loop(0, nchunk // 2 - 2, body, 0)

        g = nchunk - 4
        pltpu.make_async_copy(table_hbm.at[idx0], rows0, sem0).wait()
        pltpu.sync_copy(rows0, out_hbm.at[pl.ds(base + g * CHUNK, CHUNK)])
        pltpu.sync_copy(idx_hbm.at[pl.ds(base + (g + 2) * CHUNK, CHUNK)], idx0)
        pltpu.async_copy(table_hbm.at[idx0], rows0, sem0)
        pltpu.make_async_copy(table_hbm.at[idx1], rows1, sem1).wait()
        pltpu.sync_copy(rows1, out_hbm.at[pl.ds(base + (g + 1) * CHUNK, CHUNK)])
        pltpu.sync_copy(idx_hbm.at[pl.ds(base + (g + 3) * CHUNK, CHUNK)], idx1)
        pltpu.async_copy(table_hbm.at[idx1], rows1, sem1)

        pltpu.make_async_copy(table_hbm.at[idx0], rows0, sem0).wait()
        pltpu.sync_copy(rows0,
                        out_hbm.at[pl.ds(base + (g + 2) * CHUNK, CHUNK)])
        pltpu.make_async_copy(table_hbm.at[idx1], rows1, sem1).wait()
        pltpu.sync_copy(rows1,
                        out_hbm.at[pl.ds(base + (g + 3) * CHUNK, CHUNK)])

    return gather_kernel(table, idx)


def kernel(phoneme_indices, phoneme_table, sutra_table, position_table,
           sutra_lookup, position_lookup, proj_w, proj_b):
    batch, seq = phoneme_indices.shape
    fused = _build_fused_table(phoneme_table, sutra_table, position_table,
                               sutra_lookup, position_lookup, proj_w, proj_b)
    idx = phoneme_indices.reshape(-1).astype(jnp.int32)
    out = _sc_gather(fused, idx, batch * seq)
    return out[:, :D].reshape(batch, seq, D)
